# Initial kernel scaffold; baseline (speedup 1.0000x reference)
import jax
import jax.numpy as jnp
from jax.experimental import pallas as pl


def kernel(embeddings, batch_id):
    raise NotImplementedError("write your pallas kernel here")


# speedup vs baseline: 6.6542x; 6.6542x over previous
import jax
import jax.numpy as jnp
from jax.experimental import pallas as pl
from jax.experimental.pallas import tpu as pltpu

_K = 64
_DIM = 128
_NITER = 25
_N = 4096


def _body(emb_ref, cent0_ref, out_ref):
    emb = emb_ref[...]
    enorm = jnp.sum(emb * emb, axis=1, keepdims=True)
    cols = jax.lax.broadcasted_iota(jnp.int32, (_N, _K), 1)

    def dists(cent):
        cnorm = jnp.sum(cent * cent, axis=1)
        g = jax.lax.dot_general(
            emb, cent, (((1,), (1,)), ((), ())),
            preferred_element_type=jnp.float32)
        return enorm - 2.0 * g + cnorm[None, :]

    def first_eq_idx(x, target):
        return jnp.min(jnp.where(x == target, cols, _K), axis=1, keepdims=True)

    def step(_, cent):
        sq = dists(cent)
        mn = jnp.min(sq, axis=1, keepdims=True)
        idx = first_eq_idx(sq, mn)
        onehot = (idx == cols).astype(jnp.float32)
        sums = jax.lax.dot_general(
            onehot, emb, (((0,), (0,)), ((), ())),
            preferred_element_type=jnp.float32,
            precision=jax.lax.Precision.HIGHEST)
        counts = jnp.sum(onehot, axis=0)[:, None]
        return jnp.where(counts > 0, sums / jnp.maximum(counts, 1.0), cent)

    cent = jax.lax.fori_loop(0, _NITER, step, cent0_ref[...])

    sq = dists(cent)
    dist = jnp.sqrt(jnp.maximum(sq, 0.0))
    mx = jnp.max(dist, axis=1, keepdims=True)
    m1 = first_eq_idx(dist, mx)
    dist2 = jnp.where(cols == m1, -jnp.inf, dist)
    mx2 = jnp.max(dist2, axis=1, keepdims=True)
    m2 = first_eq_idx(dist2, mx2)
    onehot2 = (m2 == cols).astype(jnp.float32)
    out_ref[...] = jax.lax.dot_general(
        onehot2, cent, (((1,), (0,)), ((), ())),
        preferred_element_type=jnp.float32,
        precision=jax.lax.Precision.HIGHEST)


def kernel(embeddings, batch_id):
    del batch_id
    perm = jax.random.permutation(jax.random.key(42), embeddings.shape[0])
    cent0 = embeddings[perm[:_K]]
    return pl.pallas_call(
        _body,
        out_shape=jax.ShapeDtypeStruct((_N, _DIM), jnp.float32),
    )(embeddings, cent0)


# speedup vs baseline: 22.5519x; 3.3891x over previous
import jax
import jax.numpy as jnp
from jax.experimental import pallas as pl
from jax.experimental.pallas import tpu as pltpu

_K = 64
_DIM = 128
_NITER = 25
_N = 4096


def _body(emb_ref, cent0_ref, out_ref):
    emb = emb_ref[...]
    emb_t = jnp.swapaxes(emb, 0, 1)
    enorm = jnp.swapaxes(
        jnp.sum(emb * emb, axis=1, keepdims=True), 0, 1)
    rows = jax.lax.broadcasted_iota(
        jnp.int32, (_K, _N), 0).astype(jnp.float32)

    def dists_t(cent):
        cnorm = jnp.sum(cent * cent, axis=1, keepdims=True)
        g = jax.lax.dot_general(
            cent, emb_t, (((1,), (0,)), ((), ())),
            preferred_element_type=jnp.float32)
        return enorm - 2.0 * g + cnorm

    def first_eq_idx(x, target):
        return jnp.min(jnp.where(x == target, rows, float(_K)),
                       axis=0, keepdims=True)

    def step(_, cent):
        sq = dists_t(cent)
        mn = jnp.min(sq, axis=0, keepdims=True)
        idx = first_eq_idx(sq, mn)
        onehot_t = (idx == rows).astype(jnp.float32)
        sums = jax.lax.dot_general(
            onehot_t, emb, (((1,), (0,)), ((), ())),
            preferred_element_type=jnp.float32,
            precision=jax.lax.Precision.HIGHEST)
        counts = jnp.sum(onehot_t, axis=1, keepdims=True)
        return jnp.where(counts > 0, sums / jnp.maximum(counts, 1.0), cent)

    cent = jax.lax.fori_loop(0, _NITER, step, cent0_ref[...])

    sq = dists_t(cent)
    dist = jnp.sqrt(jnp.maximum(sq, 0.0))
    mx = jnp.max(dist, axis=0, keepdims=True)
    m1 = first_eq_idx(dist, mx)
    dist2 = jnp.where(rows == m1, -jnp.inf, dist)
    mx2 = jnp.max(dist2, axis=0, keepdims=True)
    m2 = first_eq_idx(dist2, mx2)
    cols = jax.lax.broadcasted_iota(
        jnp.int32, (_N, _K), 1).astype(jnp.float32)
    onehot2 = (jnp.swapaxes(m2, 0, 1) == cols).astype(jnp.float32)
    out_ref[...] = jax.lax.dot_general(
        onehot2, cent, (((1,), (0,)), ((), ())),
        preferred_element_type=jnp.float32,
        precision=jax.lax.Precision.HIGHEST)


def kernel(embeddings, batch_id):
    del batch_id
    perm = jax.random.permutation(jax.random.key(42), embeddings.shape[0])
    cent0 = embeddings[perm[:_K]]
    return pl.pallas_call(
        _body,
        out_shape=jax.ShapeDtypeStruct((_N, _DIM), jnp.float32),
    )(embeddings, cent0)
